# baseline (speedup 1.0000x reference)
import jax
import jax.numpy as jnp
from jax.experimental import pallas as pl
from jax.experimental.pallas import tpu as pltpu

_D0, _D1, _K = 2048, 2048, 4
_Q, _C = 64, 128
_BI = 256
_BQ = 8
_NI = _D0 // _BI
_NQ = _Q // _BQ


def _mask_kernel(x_ref, o_ref):
    bi = pl.program_id(0)
    bq = pl.program_id(1)

    @pl.when(bq < bi)
    def _zero():
        o_ref[...] = jnp.zeros_like(o_ref)

    @pl.when(bq > bi)
    def _copy():
        o_ref[...] = x_ref[...]

    @pl.when(bq == bi)
    def _diag():
        rows = jax.lax.broadcasted_iota(jnp.int32, (_BI, _BQ, _C), 0) + bi * _BI
        qs = jax.lax.broadcasted_iota(jnp.int32, (_BI, _BQ, _C), 1) + bq * _BQ
        cs = jax.lax.broadcasted_iota(jnp.int32, (_BI, _BQ, _C), 2)
        keep = (qs >> 2) * _C + cs >= rows
        o_ref[...] = jnp.where(keep, x_ref[...], 0.0)


def _in_index(bi, bq):
    return (bi, bq, 0)


def kernel(w):
    x = (w.reshape(_D0, 16, _C, _K)
          .transpose(0, 1, 3, 2)
          .reshape(_D0, _Q, _C))
    out = pl.pallas_call(
        _mask_kernel,
        grid=(_NI, _NQ),
        in_specs=[pl.BlockSpec((_BI, _BQ, _C), _in_index)],
        out_specs=pl.BlockSpec((_BI, _BQ, _C), lambda bi, bq: (bi, bq, 0)),
        out_shape=jax.ShapeDtypeStruct((_D0, _Q, _C), jnp.float32),
    )(x)
    return (out.reshape(_D0, 16, _K, _C)
               .transpose(0, 1, 3, 2)
               .reshape(_D0, _D1, _K))


# speedup vs baseline: 1.6793x; 1.6793x over previous
import jax
import jax.numpy as jnp
from jax.experimental import pallas as pl
from jax.experimental.pallas import tpu as pltpu

_D0, _D1, _K = 2048, 2048, 4
_Q, _C = 64, 128
_BI = 512
_BQ = 16
_NI = _D0 // _BI
_NQ = _Q // _BQ


def _mask_kernel(x_ref, o_ref):
    bi = pl.program_id(0)
    bq = pl.program_id(1)

    @pl.when(bq < bi)
    def _zero():
        o_ref[...] = jnp.zeros_like(o_ref)

    @pl.when(bq > bi)
    def _copy():
        o_ref[...] = x_ref[...]

    @pl.when(bq == bi)
    def _diag():
        rows = jax.lax.broadcasted_iota(jnp.int32, (_BI, _BQ, _C), 0) + bi * _BI
        qs = jax.lax.broadcasted_iota(jnp.int32, (_BI, _BQ, _C), 1) + bq * _BQ
        cs = jax.lax.broadcasted_iota(jnp.int32, (_BI, _BQ, _C), 2)
        keep = (qs >> 2) * _C + cs >= rows
        o_ref[...] = jnp.where(keep, x_ref[...], 0.0)


def _in_index(bi, bq):
    is_zero = bq < bi
    return (jnp.where(is_zero, bi - 1, bi), jnp.where(is_zero, _NQ - 1, bq), 0)


def kernel(w):
    x = (w.reshape(_D0, 16, _C, _K)
          .transpose(0, 1, 3, 2)
          .reshape(_D0, _Q, _C))
    out = pl.pallas_call(
        _mask_kernel,
        grid=(_NI, _NQ),
        in_specs=[pl.BlockSpec((_BI, _BQ, _C), _in_index)],
        out_specs=pl.BlockSpec((_BI, _BQ, _C), lambda bi, bq: (bi, bq, 0)),
        out_shape=jax.ShapeDtypeStruct((_D0, _Q, _C), jnp.float32),
    )(x)
    return (out.reshape(_D0, 16, _K, _C)
               .transpose(0, 1, 3, 2)
               .reshape(_D0, _D1, _K))


# speedup vs baseline: 2.0596x; 1.2265x over previous
import jax
import jax.numpy as jnp
from jax.experimental import pallas as pl
from jax.experimental.pallas import tpu as pltpu

_D0, _D1, _K = 2048, 2048, 4
_Q, _C = 64, 128
_BI = 256
_NI = _D0 // _BI
_BQ = 8
_NQ = _Q // _BQ


def _issue_copies(x_hbm, scr, sem, bi):
    slot = jax.lax.rem(bi, 2)
    row0 = bi * _BI

    def body(cq, _):
        @pl.when(cq >= bi)
        def _():
            pltpu.make_async_copy(
                x_hbm.at[pl.ds(row0, _BI), pl.ds(cq * _BQ, _BQ), :],
                scr.at[slot, :, pl.ds(cq * _BQ, _BQ), :],
                sem.at[slot, cq],
            ).start()
        return 0

    jax.lax.fori_loop(0, _NQ, body, 0)


def _wait_copies(x_hbm, scr, sem, bi):
    slot = jax.lax.rem(bi, 2)
    row0 = bi * _BI

    def body(cq, _):
        @pl.when(cq >= bi)
        def _():
            pltpu.make_async_copy(
                x_hbm.at[pl.ds(row0, _BI), pl.ds(cq * _BQ, _BQ), :],
                scr.at[slot, :, pl.ds(cq * _BQ, _BQ), :],
                sem.at[slot, cq],
            ).wait()
        return 0

    jax.lax.fori_loop(0, _NQ, body, 0)


def _mask_kernel(x_hbm, o_ref, scr, sem):
    bi = pl.program_id(0)

    @pl.when(bi == 0)
    def _prologue():
        _issue_copies(x_hbm, scr, sem, 0)

    @pl.when(bi + 1 < _NI)
    def _prefetch():
        _issue_copies(x_hbm, scr, sem, bi + 1)

    _wait_copies(x_hbm, scr, sem, bi)

    slot = jax.lax.rem(bi, 2)
    rows = jax.lax.broadcasted_iota(jnp.int32, (_BI, _Q, _C), 0) + bi * _BI
    qs = jax.lax.broadcasted_iota(jnp.int32, (_BI, _Q, _C), 1)
    cs = jax.lax.broadcasted_iota(jnp.int32, (_BI, _Q, _C), 2)
    keep = (qs >> 2) * _C + cs >= rows
    o_ref[...] = jnp.where(keep, scr[slot], 0.0)


def kernel(w):
    x = (w.reshape(_D0, 16, _C, _K)
          .transpose(0, 1, 3, 2)
          .reshape(_D0, _Q, _C))
    out = pl.pallas_call(
        _mask_kernel,
        grid=(_NI,),
        in_specs=[pl.BlockSpec(memory_space=pltpu.MemorySpace.HBM)],
        out_specs=pl.BlockSpec((_BI, _Q, _C), lambda bi: (bi, 0, 0)),
        out_shape=jax.ShapeDtypeStruct((_D0, _Q, _C), jnp.float32),
        scratch_shapes=[
            pltpu.VMEM((2, _BI, _Q, _C), jnp.float32),
            pltpu.SemaphoreType.DMA((2, _NQ)),
        ],
    )(x)
    return (out.reshape(_D0, 16, _K, _C)
               .transpose(0, 1, 3, 2)
               .reshape(_D0, _D1, _K))
